# baseline (speedup 1.0000x reference)
import jax
import jax.numpy as jnp
from jax.experimental import pallas as pl
from jax.experimental.pallas import tpu as pltpu


def _matmul_bias_kernel(x_ref, w_ref, b_ref, o_ref):
    x16 = x_ref[...].astype(jnp.bfloat16)
    w16 = w_ref[...].astype(jnp.bfloat16)
    acc = jnp.dot(x16, w16, preferred_element_type=jnp.float32)
    o_ref[...] = (acc + b_ref[...]).astype(o_ref.dtype)


def kernel(x, weight, bias):
    out_dtype = x.dtype
    lead_shape = x.shape[:-1]
    d_in = x.shape[-1]
    d_out = weight.shape[1]
    x2 = x.reshape(-1, d_in)
    b_rows = x2.shape[0]

    b2d = bias.astype(jnp.float32).reshape(1, d_out)

    tile_b = min(512, b_rows)
    n_phases = 2
    tile_n = d_out // n_phases
    grid = (n_phases, pl.cdiv(b_rows, tile_b))

    out = pl.pallas_call(
        _matmul_bias_kernel,
        out_shape=jax.ShapeDtypeStruct((b_rows, d_out), out_dtype),
        grid=grid,
        in_specs=[
            pl.BlockSpec((tile_b, d_in), lambda j, i: (i, 0)),
            pl.BlockSpec((d_in, tile_n), lambda j, i: (0, j)),
            pl.BlockSpec((1, tile_n), lambda j, i: (0, j)),
        ],
        out_specs=pl.BlockSpec((tile_b, tile_n), lambda j, i: (i, j)),
        compiler_params=pltpu.CompilerParams(
            dimension_semantics=("arbitrary", "arbitrary"),
            vmem_limit_bytes=96 * 1024 * 1024,
        ),
        cost_estimate=pl.CostEstimate(
            flops=2 * b_rows * d_in * d_out,
            transcendentals=0,
            bytes_accessed=(2 * x2.size * 4 + weight.size * 4
                            + b_rows * d_out * 4 + d_out * 4),
        ),
    )(x2, weight, b2d)

    return out.reshape(lead_shape + (d_out,))


# speedup vs baseline: 1.0438x; 1.0438x over previous
import functools

import jax
import jax.numpy as jnp
from jax import lax
from jax.experimental import pallas as pl
from jax.experimental.pallas import tpu as pltpu

_TILE = 512
_WCHUNK = 256


def _mm_kernel(x_hbm, w_hbm, b_ref, o_hbm,
               xbuf, wst, w16, obuf, xsem, wsem, osem,
               *, n_tiles, n_wchunks):
    d_in = w_hbm.shape[0]
    d_out = w_hbm.shape[1]

    def xcopy(t, slot):
        return pltpu.make_async_copy(
            x_hbm.at[pl.ds(t * _TILE, _TILE), :], xbuf.at[slot], xsem.at[slot])

    def wcopy(c, slot):
        return pltpu.make_async_copy(
            w_hbm.at[pl.ds(c * _WCHUNK, _WCHUNK), :], wst.at[slot],
            wsem.at[slot])

    def ocopy(t, slot):
        return pltpu.make_async_copy(
            obuf.at[slot], o_hbm.at[pl.ds(t * _TILE, _TILE), :], osem.at[slot])

    wcopy(0, 0).start()
    wcopy(1, 1).start()
    xcopy(0, 0).start()
    xcopy(1, 1).start()

    xcopy(0, 0).wait()
    x16 = xbuf[0].astype(jnp.bfloat16)
    acc = None
    for c in range(n_wchunks):
        wcopy(c, c % 2).wait()
        sl = slice(c * _WCHUNK, (c + 1) * _WCHUNK)
        w16[sl, :] = wst[c % 2].astype(jnp.bfloat16)
        if c + 2 < n_wchunks:
            wcopy(c + 2, c % 2).start()
        part = jnp.dot(x16[:, sl], w16[sl, :],
                       preferred_element_type=jnp.float32)
        acc = part if acc is None else acc + part
    obuf[0] = acc + b_ref[...]
    ocopy(0, 0).start()

    for t in range(1, n_tiles):
        slot = t % 2
        nxt = (t + 1) % 2
        if t + 1 < n_tiles:
            xcopy(t + 1, nxt).start()
        xcopy(t, slot).wait()
        y = jnp.dot(xbuf[slot].astype(jnp.bfloat16), w16[...],
                    preferred_element_type=jnp.float32) + b_ref[...]
        if t >= 2:
            ocopy(t - 2, slot).wait()
        obuf[slot] = y
        ocopy(t, slot).start()

    ocopy(n_tiles - 2, (n_tiles - 2) % 2).wait()
    ocopy(n_tiles - 1, (n_tiles - 1) % 2).wait()


def kernel(x, weight, bias):
    out_dtype = x.dtype
    lead_shape = x.shape[:-1]
    d_in = x.shape[-1]
    d_out = weight.shape[1]
    x2 = x.reshape(-1, d_in)
    b_rows = x2.shape[0]
    n_tiles = b_rows // _TILE
    n_wchunks = d_in // _WCHUNK

    b2d = bias.astype(jnp.float32).reshape(1, d_out)

    out = pl.pallas_call(
        functools.partial(_mm_kernel, n_tiles=n_tiles, n_wchunks=n_wchunks),
        out_shape=jax.ShapeDtypeStruct((b_rows, d_out), out_dtype),
        in_specs=[
            pl.BlockSpec(memory_space=pltpu.HBM),
            pl.BlockSpec(memory_space=pltpu.HBM),
            pl.BlockSpec((1, d_out), lambda: (0, 0)),
        ],
        out_specs=pl.BlockSpec(memory_space=pltpu.HBM),
        scratch_shapes=[
            pltpu.VMEM((2, _TILE, d_in), jnp.float32),
            pltpu.VMEM((2, _WCHUNK, d_out), jnp.float32),
            pltpu.VMEM((d_in, d_out), jnp.bfloat16),
            pltpu.VMEM((2, _TILE, d_out), jnp.float32),
            pltpu.SemaphoreType.DMA((2,)),
            pltpu.SemaphoreType.DMA((2,)),
            pltpu.SemaphoreType.DMA((2,)),
        ],
        compiler_params=pltpu.CompilerParams(
            dimension_semantics=(),
            vmem_limit_bytes=96 * 1024 * 1024,
        ),
    )(x2, weight, b2d)

    return out.reshape(lead_shape + (d_out,))


# speedup vs baseline: 1.1308x; 1.0833x over previous
import functools

import jax
import jax.numpy as jnp
from jax.experimental import pallas as pl
from jax.experimental.pallas import tpu as pltpu

_WCHUNK = 256


def _mm_kernel(x_ref, w_hbm, b_ref, o_ref, wst, w16, wsem, *, n_wchunks):
    def wcopy(c, slot):
        return pltpu.make_async_copy(
            w_hbm.at[pl.ds(c * _WCHUNK, _WCHUNK), :], wst.at[slot],
            wsem.at[slot])

    x16 = x_ref[...].astype(jnp.bfloat16)

    @pl.when(pl.program_id(0) == 0)
    def _():
        wcopy(0, 0).start()
        wcopy(1, 1).start()
        acc = jnp.zeros_like(o_ref)
        for c in range(n_wchunks):
            wcopy(c, c % 2).wait()
            sl = slice(c * _WCHUNK, (c + 1) * _WCHUNK)
            w16[sl, :] = wst[c % 2].astype(jnp.bfloat16)
            if c + 2 < n_wchunks:
                wcopy(c + 2, c % 2).start()
            acc = acc + jnp.dot(x16[:, sl], w16[sl, :],
                                preferred_element_type=jnp.float32)
        o_ref[...] = (acc + b_ref[...]).astype(o_ref.dtype)

    @pl.when(pl.program_id(0) > 0)
    def _():
        acc = jnp.dot(x16, w16[...], preferred_element_type=jnp.float32)
        o_ref[...] = (acc + b_ref[...]).astype(o_ref.dtype)


def kernel(x, weight, bias):
    out_dtype = x.dtype
    lead_shape = x.shape[:-1]
    d_in = x.shape[-1]
    d_out = weight.shape[1]
    x2 = x.reshape(-1, d_in)
    b_rows = x2.shape[0]
    n_wchunks = d_in // _WCHUNK

    b2d = bias.astype(jnp.float32).reshape(1, d_out)

    tile_b = min(512, b_rows)
    grid = (pl.cdiv(b_rows, tile_b),)

    out = pl.pallas_call(
        functools.partial(_mm_kernel, n_wchunks=n_wchunks),
        out_shape=jax.ShapeDtypeStruct((b_rows, d_out), out_dtype),
        grid=grid,
        in_specs=[
            pl.BlockSpec((tile_b, d_in), lambda i: (i, 0)),
            pl.BlockSpec(memory_space=pltpu.HBM),
            pl.BlockSpec((1, d_out), lambda i: (0, 0)),
        ],
        out_specs=pl.BlockSpec((tile_b, d_out), lambda i: (i, 0)),
        scratch_shapes=[
            pltpu.VMEM((2, _WCHUNK, d_out), jnp.float32),
            pltpu.VMEM((d_in, d_out), jnp.bfloat16),
            pltpu.SemaphoreType.DMA((2,)),
        ],
        compiler_params=pltpu.CompilerParams(
            dimension_semantics=("arbitrary",),
            vmem_limit_bytes=96 * 1024 * 1024,
        ),
        cost_estimate=pl.CostEstimate(
            flops=2 * b_rows * d_in * d_out,
            transcendentals=0,
            bytes_accessed=(x2.size * 4 + weight.size * 4
                            + b_rows * d_out * 4 + d_out * 4),
        ),
    )(x2, weight, b2d)

    return out.reshape(lead_shape + (d_out,))


# speedup vs baseline: 1.1491x; 1.0162x over previous
import jax
import jax.numpy as jnp
from jax.experimental import pallas as pl
from jax.experimental.pallas import tpu as pltpu


def _matmul_bias_kernel(x_ref, w_ref, b_ref, o_ref):
    x16 = x_ref[...].astype(jnp.bfloat16)
    w16 = w_ref[...].astype(jnp.bfloat16)
    acc = jnp.dot(x16, w16, preferred_element_type=jnp.float32)
    o_ref[...] = (acc + b_ref[...]).astype(o_ref.dtype)


def kernel(x, weight, bias):
    out_dtype = x.dtype
    lead_shape = x.shape[:-1]
    d_in = x.shape[-1]
    d_out = weight.shape[1]
    x2 = x.reshape(-1, d_in)
    b_rows = x2.shape[0]

    b2d = bias.astype(jnp.float32).reshape(1, d_out)

    tile_b = min(512, b_rows)
    grid = (pl.cdiv(b_rows, tile_b),)

    out = pl.pallas_call(
        _matmul_bias_kernel,
        out_shape=jax.ShapeDtypeStruct((b_rows, d_out), out_dtype),
        grid=grid,
        in_specs=[
            pl.BlockSpec((tile_b, d_in), lambda i: (i, 0)),
            pl.BlockSpec((d_in, d_out), lambda i: (0, 0)),
            pl.BlockSpec((1, d_out), lambda i: (0, 0)),
        ],
        out_specs=pl.BlockSpec((tile_b, d_out), lambda i: (i, 0)),
        compiler_params=pltpu.CompilerParams(
            dimension_semantics=("parallel",),
            vmem_limit_bytes=96 * 1024 * 1024,
        ),
        cost_estimate=pl.CostEstimate(
            flops=2 * b_rows * d_in * d_out,
            transcendentals=0,
            bytes_accessed=(x2.size * 4 + weight.size * 4
                            + b_rows * d_out * 4 + d_out * 4),
        ),
    )(x2, weight, b2d)

    return out.reshape(lead_shape + (d_out,))
